# Initial kernel scaffold; baseline (speedup 1.0000x reference)
import jax
import jax.numpy as jnp
from jax.experimental import pallas as pl


def kernel(xyzp, features, Wc, bc):
    raise NotImplementedError("write your pallas kernel here")


# speedup vs baseline: 41.7370x; 41.7370x over previous
import functools

import jax
import jax.numpy as jnp
from jax import lax
from jax.experimental import pallas as pl
from jax.experimental.pallas import tpu as pltpu
from jax.experimental.pallas import tpu_sc as plsc

H, W_ = 256, 256
B, N = 16, 20000
BN = B * N
CIN, COUT = 18, 32
CS = 24
Wp = 258
SB = 66688
ZROW = SB - 1
BASE = 264
M = 8192
NCHUNK = -(-(B * SB) // M)
O_ROWS = NCHUNK * M
TAIL0 = BASE + B * SB
G_ROWS = TAIL0 + 32 * 208
WIN = 1040
SUB = 512
PPW = 1280
GCH = 128
NGC = 79
PTS_W = 10000

_OFFS = tuple(dy * Wp + dx for dy in (-1, 0, 1) for dx in (-1, 0, 1))

_mesh = plsc.VectorSubcoreMesh(core_axis_name="c", subcore_axis_name="s")
_sc_params = pltpu.CompilerParams(use_tc_tiling_on_sc=False)


@functools.partial(
    pl.kernel,
    out_type=jax.ShapeDtypeStruct((G_ROWS, CS), jnp.float32),
    mesh=_mesh,
    compiler_params=_sc_params,
    scratch_types=[
        pltpu.VMEM_SHARED((SB, CS), jnp.float32),
        pltpu.VMEM((256, CS), jnp.float32),
        pltpu.VMEM((640, CS), jnp.float32),
        pltpu.VMEM((10, 128), jnp.int32),
    ],
)
def _sc_scatter(rows_hbm, lkey_hbm, g_hbm, sg, zbuf, rbuf, kidx):
    c = lax.axis_index("c")
    s = lax.axis_index("s")
    w = s * 2 + c
    z16 = jnp.zeros((16,), jnp.float32)

    def _zrow(i, _):
        zbuf[i, pl.ds(0, 16)] = z16
        zbuf[i, pl.ds(8, 16)] = z16
        return 0

    lax.fori_loop(0, 256, _zrow, 0)

    pltpu.sync_copy(zbuf.at[pl.ds(0, 208)], g_hbm.at[pl.ds(TAIL0 + w * 208, 208)])

    @pl.when(w == 0)
    def _():
        pltpu.sync_copy(zbuf, g_hbm.at[pl.ds(0, 256)])
        pltpu.sync_copy(zbuf.at[pl.ds(0, 8)], g_hbm.at[pl.ds(256, 8)])

    for j in range(8):
        b = c * 8 + j
        def _zchunk(q, _):
            pltpu.sync_copy(zbuf, sg.at[pl.ds(s * 4168 + q * 256, 256)])
            return 0

        lax.fori_loop(0, 16, _zchunk, 0)
        pltpu.sync_copy(zbuf.at[pl.ds(0, 72)], sg.at[pl.ds(s * 4168 + 4096, 72)])
        plsc.subcore_barrier()
        widx = b * 16 + s
        pltpu.sync_copy(lkey_hbm.at[widx], kidx)
        for h in range(2):
            pltpu.sync_copy(rows_hbm.at[widx, h], rbuf)
            for t in range(5):
                pltpu.sync_copy(rbuf.at[pl.ds(t * 128, 128)],
                                sg.at[kidx.at[h * 5 + t]], add=True)
        plsc.subcore_barrier()
        pltpu.sync_copy(sg.at[pl.ds(s * 4168, 4168)],
                        g_hbm.at[pl.ds(BASE + b * SB + s * 4168, 4168)])


def _conv_body(g_any, w_ref, bc_ref, o_ref, buf, ab, sem):
    g = pl.program_id(0)

    def cp(i):
        return pltpu.make_async_copy(
            g_any.at[pl.ds(i * M, M + 528)], buf.at[i % 2], sem.at[i % 2])

    @pl.when(g == 0)
    def _():
        cp(0).start()

    @pl.when(g + 1 < NCHUNK)
    def _():
        cp(g + 1).start()

    cp(g).wait()
    wmat = w_ref[...].astype(jnp.bfloat16)
    bias = bc_ref[0:1, :]
    for i in range(10):
        win = buf[g % 2, pl.ds(i * 872, 872), :]
        cnt = jnp.maximum(win[:, 0:1] + win[:, 1:2], 1.0)
        ab[pl.ds(i * 872, 872), :] = (win * (1.0 / cnt)).astype(jnp.bfloat16)
    z8 = jnp.zeros((SUB, 32 - CS), jnp.bfloat16)
    for i in range(M // SUB):
        pieces = []
        for o in _OFFS:
            pieces.append(ab[pl.ds(i * SUB + 264 + o, SUB), :])
            pieces.append(z8)
        acat = jnp.concatenate(pieces, axis=1)
        res = lax.dot_general(acat, wmat, (((1,), (0,)), ((), ())),
                              preferred_element_type=jnp.float32)
        o_ref[pl.ds(i * SUB, SUB), :] = res + bias


def _tc_conv(g_grid, wcat, bc8):
    return pl.pallas_call(
        _conv_body,
        grid=(NCHUNK,),
        in_specs=[
            pl.BlockSpec(memory_space=pltpu.HBM),
            pl.BlockSpec((9 * 32, COUT), lambda g: (0, 0)),
            pl.BlockSpec((8, COUT), lambda g: (0, 0)),
        ],
        out_specs=pl.BlockSpec((M, COUT), lambda g: (g, 0)),
        out_shape=jax.ShapeDtypeStruct((O_ROWS, COUT), jnp.float32),
        scratch_shapes=[
            pltpu.VMEM((2, M + 528, CS), jnp.float32),
            pltpu.VMEM((M + 528, CS), jnp.bfloat16),
            pltpu.SemaphoreType.DMA((2,)),
        ],
    )(g_grid, wcat, bc8)


@functools.partial(
    pl.kernel,
    out_type=jax.ShapeDtypeStruct((BN, COUT), jnp.float32),
    mesh=_mesh,
    compiler_params=_sc_params,
    scratch_types=[
        pltpu.VMEM((NGC, GCH), jnp.int32),
        pltpu.VMEM((GCH, COUT), jnp.float32),
    ],
)
def _sc_gather(o_hbm, gkey_hbm, out_hbm, kbuf, obuf):
    c = lax.axis_index("c")
    s = lax.axis_index("s")
    w = s * 2 + c
    base = w * PTS_W
    pltpu.sync_copy(gkey_hbm.at[w], kbuf)

    def _chunk(t, _):
        pltpu.sync_copy(o_hbm.at[kbuf.at[t]], obuf)
        pltpu.sync_copy(obuf, out_hbm.at[pl.ds(base + t * GCH, GCH)])
        return 0

    lax.fori_loop(0, NGC - 1, _chunk, 0)
    pltpu.sync_copy(o_hbm.at[kbuf.at[NGC - 1]], obuf)
    pltpu.sync_copy(obuf.at[pl.ds(0, 16)],
                    out_hbm.at[pl.ds(base + (NGC - 1) * GCH, 16)])


def kernel(xyzp, features, Wc, bc):
    y = jnp.clip(jnp.round(xyzp[..., 1] * H), 0, H - 1).astype(jnp.int32)
    x = jnp.clip(jnp.round(xyzp[..., 0] * W_), 0, W_ - 1).astype(jnp.int32)
    pos = xyzp[..., 3:4]
    rows = jnp.concatenate(
        [pos, 1.0 - pos, features, jnp.zeros((B, N, CS - CIN), jnp.float32)],
        axis=-1)
    lkey = (y + 1) * Wp + (x + 1)

    rows_p = jnp.concatenate(
        [rows, jnp.zeros((B, 16 * PPW - N, CS), jnp.float32)], axis=1)
    rows_p = rows_p.reshape(B * 16, 2, PPW // 2, CS)
    lkey_p = jnp.concatenate(
        [lkey, jnp.full((B, 16 * PPW - N), ZROW, jnp.int32)], axis=1)
    lkey_p = lkey_p.reshape(B * 16, 10, 128)

    g_grid = _sc_scatter(rows_p, lkey_p)

    wcat = jnp.pad(Wc, ((0, 0), (0, 32 - CIN), (0, 0))).reshape(9 * 32, COUT)
    bc8 = jnp.tile(bc[None, :], (8, 1))
    o_conv = _tc_conv(g_grid, wcat, bc8)

    gkey = (jnp.arange(B, dtype=jnp.int32)[:, None] * SB + lkey).reshape(32, PTS_W)
    gkey_p = jnp.concatenate(
        [gkey, jnp.zeros((32, NGC * GCH - PTS_W), jnp.int32)], axis=1)
    gkey_p = gkey_p.reshape(32, NGC, GCH)
    out = _sc_gather(o_conv, gkey_p)
    return out.reshape(B, N, COUT)


# speedup vs baseline: 47.7600x; 1.1443x over previous
import functools

import jax
import jax.numpy as jnp
from jax import lax
from jax.experimental import pallas as pl
from jax.experimental.pallas import tpu as pltpu
from jax.experimental.pallas import tpu_sc as plsc

H, W_ = 256, 256
B, N = 16, 20000
BN = B * N
CIN, COUT = 18, 32
CS = 24
Wp = 258
SB = 66688
ZROW = SB - 1
BASE = 264
M = 8192
NCHUNK = -(-(B * SB) // M)
O_ROWS = NCHUNK * M
TAIL0 = BASE + B * SB
G_ROWS = TAIL0 + 32 * 208
WIN = 1040
SUB = 512
PPW = 1280
GCH = 128
NGC = 79
PTS_W = 10000

_OFFS = tuple(dy * Wp + dx for dy in (-1, 0, 1) for dx in (-1, 0, 1))

_mesh = plsc.VectorSubcoreMesh(core_axis_name="c", subcore_axis_name="s")
_sc_params = pltpu.CompilerParams(use_tc_tiling_on_sc=False)


@functools.partial(
    pl.kernel,
    out_type=jax.ShapeDtypeStruct((G_ROWS, CS), jnp.float32),
    mesh=_mesh,
    compiler_params=_sc_params,
    scratch_types=[
        pltpu.VMEM_SHARED((SB, CS), jnp.float32),
        pltpu.VMEM((256, CS), jnp.float32),
        pltpu.VMEM((640, CS), jnp.float32),
        pltpu.VMEM((10, 128), jnp.int32),
    ],
)
def _sc_scatter(rows_hbm, lkey_hbm, g_hbm, sg, zbuf, rbuf, kidx):
    c = lax.axis_index("c")
    s = lax.axis_index("s")
    w = s * 2 + c
    z16 = jnp.zeros((16,), jnp.float32)

    def _zrow(i, _):
        zbuf[i, pl.ds(0, 16)] = z16
        zbuf[i, pl.ds(8, 16)] = z16
        return 0

    lax.fori_loop(0, 256, _zrow, 0)

    pltpu.sync_copy(zbuf.at[pl.ds(0, 208)], g_hbm.at[pl.ds(TAIL0 + w * 208, 208)])

    @pl.when(w == 0)
    def _():
        pltpu.sync_copy(zbuf, g_hbm.at[pl.ds(0, 256)])
        pltpu.sync_copy(zbuf.at[pl.ds(0, 8)], g_hbm.at[pl.ds(256, 8)])

    for j in range(8):
        b = c * 8 + j
        def _zchunk(q, _):
            pltpu.sync_copy(zbuf, sg.at[pl.ds(s * 4168 + q * 256, 256)])
            return 0

        lax.fori_loop(0, 16, _zchunk, 0)
        pltpu.sync_copy(zbuf.at[pl.ds(0, 72)], sg.at[pl.ds(s * 4168 + 4096, 72)])
        plsc.subcore_barrier()
        widx = b * 16 + s
        pltpu.sync_copy(lkey_hbm.at[widx], kidx)
        for h in range(2):
            pltpu.sync_copy(rows_hbm.at[widx, h], rbuf)
            for t in range(5):
                pltpu.sync_copy(rbuf.at[pl.ds(t * 128, 128)],
                                sg.at[kidx.at[h * 5 + t]], add=True)
        plsc.subcore_barrier()
        pltpu.sync_copy(sg.at[pl.ds(s * 4168, 4168)],
                        g_hbm.at[pl.ds(BASE + b * SB + s * 4168, 4168)])


def _conv_body(g_any, w_ref, bc_ref, o_ref, buf, ab, sem):
    g = pl.program_id(0)

    def cp(i):
        return pltpu.make_async_copy(
            g_any.at[pl.ds(i * M, M + 528)], buf.at[i % 2], sem.at[i % 2])

    @pl.when(g == 0)
    def _():
        cp(0).start()

    @pl.when(g + 1 < NCHUNK)
    def _():
        cp(g + 1).start()

    cp(g).wait()
    wmat = w_ref[...].astype(jnp.bfloat16)
    bias = bc_ref[0:1, :]
    dn = (((1,), (0,)), ((), ()))
    for i in range(10):
        win = buf[g % 2, pl.ds(i * 872, 872), :]
        cnt = jnp.maximum(win[:, 0:1] + win[:, 1:2], 1.0)
        ab[pl.ds(i * 872, 872), :] = (win * (1.0 / cnt)).astype(jnp.bfloat16)
    z8 = jnp.zeros((SUB, 32 - CS), jnp.bfloat16)
    for i in range(M // SUB):
        pieces = []
        for o in _OFFS:
            pieces.append(ab[pl.ds(i * SUB + 264 + o, SUB), :])
            pieces.append(z8)
        acat = jnp.concatenate(pieces, axis=1)
        res = bias
        for q in range(4):
            res = res + lax.dot_general(
                acat[q * 128:(q + 1) * 128],
                wmat[q * 288:(q + 1) * 288],
                dn, preferred_element_type=jnp.float32)
        o_ref[pl.ds(i * SUB // 4, SUB // 4), :] = res


def _tc_conv(g_grid, wcat, bc8):
    return pl.pallas_call(
        _conv_body,
        grid=(NCHUNK,),
        in_specs=[
            pl.BlockSpec(memory_space=pltpu.HBM),
            pl.BlockSpec((4 * 9 * 32, 128), lambda g: (0, 0)),
            pl.BlockSpec((8, 128), lambda g: (0, 0)),
        ],
        out_specs=pl.BlockSpec((M // 4, 128), lambda g: (g, 0)),
        out_shape=jax.ShapeDtypeStruct((O_ROWS // 4, 128), jnp.float32),
        scratch_shapes=[
            pltpu.VMEM((2, M + 528, CS), jnp.float32),
            pltpu.VMEM((M + 528, CS), jnp.bfloat16),
            pltpu.SemaphoreType.DMA((2,)),
        ],
    )(g_grid, wcat, bc8)


@functools.partial(
    pl.kernel,
    out_type=jax.ShapeDtypeStruct((BN, COUT), jnp.float32),
    mesh=_mesh,
    compiler_params=_sc_params,
    scratch_types=[
        pltpu.VMEM((NGC, GCH), jnp.int32),
        pltpu.VMEM((GCH, COUT), jnp.float32),
    ],
)
def _sc_gather(o_hbm, gkey_hbm, out_hbm, kbuf, obuf):
    c = lax.axis_index("c")
    s = lax.axis_index("s")
    w = s * 2 + c
    base = w * PTS_W
    pltpu.sync_copy(gkey_hbm.at[w], kbuf)

    def _chunk(t, _):
        pltpu.sync_copy(o_hbm.at[kbuf.at[t]], obuf)
        pltpu.sync_copy(obuf, out_hbm.at[pl.ds(base + t * GCH, GCH)])
        return 0

    lax.fori_loop(0, NGC - 1, _chunk, 0)
    pltpu.sync_copy(o_hbm.at[kbuf.at[NGC - 1]], obuf)
    pltpu.sync_copy(obuf.at[pl.ds(0, 16)],
                    out_hbm.at[pl.ds(base + (NGC - 1) * GCH, 16)])


def kernel(xyzp, features, Wc, bc):
    y = jnp.clip(jnp.round(xyzp[..., 1] * H), 0, H - 1).astype(jnp.int32)
    x = jnp.clip(jnp.round(xyzp[..., 0] * W_), 0, W_ - 1).astype(jnp.int32)
    pos = xyzp[..., 3:4]
    rows = jnp.concatenate(
        [pos, 1.0 - pos, features, jnp.zeros((B, N, CS - CIN), jnp.float32)],
        axis=-1)
    lkey = (y + 1) * Wp + (x + 1)

    rows_p = jnp.concatenate(
        [rows, jnp.zeros((B, 16 * PPW - N, CS), jnp.float32)], axis=1)
    rows_p = rows_p.reshape(B * 16, 2, PPW // 2, CS)
    lkey_p = jnp.concatenate(
        [lkey, jnp.full((B, 16 * PPW - N), ZROW, jnp.int32)], axis=1)
    lkey_p = lkey_p.reshape(B * 16, 10, 128)

    g_grid = _sc_scatter(rows_p, lkey_p)

    wcat = jnp.pad(Wc, ((0, 0), (0, 32 - CIN), (0, 0))).reshape(9 * 32, COUT)
    wcat4 = jnp.concatenate(
        [jnp.pad(wcat, ((0, 0), (32 * q, 96 - 32 * q))) for q in range(4)], axis=0)
    bc8 = jnp.tile(bc[None, :], (8, 4))
    o_conv = _tc_conv(g_grid, wcat4, bc8).reshape(O_ROWS, COUT)

    gc = jnp.arange(B, dtype=jnp.int32)[:, None] * SB + lkey
    gkey = (4 * ((gc // 512) * 128 + gc % 128) + (gc // 128) % 4).reshape(32, PTS_W)
    gkey_p = jnp.concatenate(
        [gkey, jnp.zeros((32, NGC * GCH - PTS_W), jnp.int32)], axis=1)
    gkey_p = gkey_p.reshape(32, NGC, GCH)
    out = _sc_gather(o_conv, gkey_p)
    return out.reshape(B, N, COUT)


# speedup vs baseline: 51.5309x; 1.0790x over previous
import functools

import jax
import jax.numpy as jnp
from jax import lax
from jax.experimental import pallas as pl
from jax.experimental.pallas import tpu as pltpu
from jax.experimental.pallas import tpu_sc as plsc

H, W_ = 256, 256
B, N = 16, 20000
BN = B * N
CIN, COUT = 18, 32
CS = 32
Wp = 258
SB = 66688
ZROW = SB - 1
BASE = 264
M = 8192
NCHUNK = -(-(B * SB) // M)
O_ROWS = NCHUNK * M
TAIL0 = BASE + B * SB
G_ROWS = TAIL0 + 32 * 208
WIN = 1040
SUB = 512
PPW = 1280
GCH = 128
NGC = 79
PTS_W = 10000

_OFFS = tuple(dy * Wp + dx for dy in (-1, 0, 1) for dx in (-1, 0, 1))

_mesh = plsc.VectorSubcoreMesh(core_axis_name="c", subcore_axis_name="s")
_sc_params = pltpu.CompilerParams(use_tc_tiling_on_sc=False)


@functools.partial(
    pl.kernel,
    out_type=jax.ShapeDtypeStruct((G_ROWS, CS), jnp.bfloat16),
    mesh=_mesh,
    compiler_params=_sc_params,
    scratch_types=[
        pltpu.VMEM_SHARED((SB, CS), jnp.bfloat16),
        pltpu.VMEM((256, CS), jnp.bfloat16),
        pltpu.VMEM((640, CS), jnp.bfloat16),
        pltpu.VMEM((10, 128), jnp.int32),
    ],
)
def _sc_scatter(rows_hbm, lkey_hbm, g_hbm, sg, zbuf, rbuf, kidx):
    c = lax.axis_index("c")
    s = lax.axis_index("s")
    w = s * 2 + c
    z32 = jnp.zeros((32,), jnp.bfloat16)

    def _zrow(i, _):
        zbuf[i, pl.ds(0, 32)] = z32
        return 0

    lax.fori_loop(0, 256, _zrow, 0)

    pltpu.sync_copy(zbuf.at[pl.ds(0, 208)], g_hbm.at[pl.ds(TAIL0 + w * 208, 208)])

    @pl.when(w == 0)
    def _():
        pltpu.sync_copy(zbuf, g_hbm.at[pl.ds(0, 256)])
        pltpu.sync_copy(zbuf.at[pl.ds(0, 8)], g_hbm.at[pl.ds(256, 8)])

    for j in range(8):
        b = c * 8 + j
        def _zchunk(q, _):
            pltpu.sync_copy(zbuf, sg.at[pl.ds(s * 4168 + q * 256, 256)])
            return 0

        lax.fori_loop(0, 16, _zchunk, 0)
        pltpu.sync_copy(zbuf.at[pl.ds(0, 72)], sg.at[pl.ds(s * 4168 + 4096, 72)])
        plsc.subcore_barrier()
        widx = b * 16 + s
        pltpu.sync_copy(lkey_hbm.at[widx], kidx)
        for h in range(2):
            pltpu.sync_copy(rows_hbm.at[widx, h], rbuf)
            for t in range(5):
                pltpu.sync_copy(rbuf.at[pl.ds(t * 128, 128)],
                                sg.at[kidx.at[h * 5 + t]], add=True)
        plsc.subcore_barrier()
        pltpu.sync_copy(sg.at[pl.ds(s * 4168, 4168)],
                        g_hbm.at[pl.ds(BASE + b * SB + s * 4168, 4168)])


def _conv_body(g_any, w_ref, bc_ref, o_ref, buf, ab, sem):
    g = pl.program_id(0)

    def cp(i):
        return pltpu.make_async_copy(
            g_any.at[pl.ds(i * M, M + 528)], buf.at[i % 2], sem.at[i % 2])

    @pl.when(g == 0)
    def _():
        cp(0).start()

    @pl.when(g + 1 < NCHUNK)
    def _():
        cp(g + 1).start()

    cp(g).wait()
    wmat = w_ref[...].astype(jnp.bfloat16)
    bias = bc_ref[0:1, :]
    dn = (((1,), (0,)), ((), ()))
    for i in range(10):
        win = buf[g % 2, pl.ds(i * 872, 872), :].astype(jnp.float32)
        cnt = jnp.maximum(win[:, 0:1] + win[:, 1:2], 1.0)
        ab[pl.ds(i * 872, 872), :] = (win * (1.0 / cnt)).astype(jnp.bfloat16)
    for i in range(M // SUB):
        pieces = []
        for o in _OFFS:
            pieces.append(ab[pl.ds(i * SUB + 264 + o, SUB), :])
        acat = jnp.concatenate(pieces, axis=1)
        res = bias
        for q in range(4):
            res = res + lax.dot_general(
                acat[q * 128:(q + 1) * 128],
                wmat[q * 288:(q + 1) * 288],
                dn, preferred_element_type=jnp.float32)
        o_ref[pl.ds(i * SUB // 4, SUB // 4), :] = res


def _tc_conv(g_grid, wcat, bc8):
    return pl.pallas_call(
        _conv_body,
        grid=(NCHUNK,),
        in_specs=[
            pl.BlockSpec(memory_space=pltpu.HBM),
            pl.BlockSpec((4 * 9 * 32, 128), lambda g: (0, 0)),
            pl.BlockSpec((8, 128), lambda g: (0, 0)),
        ],
        out_specs=pl.BlockSpec((M // 4, 128), lambda g: (g, 0)),
        out_shape=jax.ShapeDtypeStruct((O_ROWS // 4, 128), jnp.float32),
        scratch_shapes=[
            pltpu.VMEM((2, M + 528, CS), jnp.bfloat16),
            pltpu.VMEM((M + 528, CS), jnp.bfloat16),
            pltpu.SemaphoreType.DMA((2,)),
        ],
    )(g_grid, wcat, bc8)


@functools.partial(
    pl.kernel,
    out_type=jax.ShapeDtypeStruct((BN, COUT), jnp.float32),
    mesh=_mesh,
    compiler_params=_sc_params,
    scratch_types=[
        pltpu.VMEM((NGC, GCH), jnp.int32),
        pltpu.VMEM((GCH, COUT), jnp.float32),
    ],
)
def _sc_gather(o_hbm, gkey_hbm, out_hbm, kbuf, obuf):
    c = lax.axis_index("c")
    s = lax.axis_index("s")
    w = s * 2 + c
    base = w * PTS_W
    pltpu.sync_copy(gkey_hbm.at[w], kbuf)

    def _chunk(t, _):
        pltpu.sync_copy(o_hbm.at[kbuf.at[t]], obuf)
        pltpu.sync_copy(obuf, out_hbm.at[pl.ds(base + t * GCH, GCH)])
        return 0

    lax.fori_loop(0, NGC - 1, _chunk, 0)
    pltpu.sync_copy(o_hbm.at[kbuf.at[NGC - 1]], obuf)
    pltpu.sync_copy(obuf.at[pl.ds(0, 16)],
                    out_hbm.at[pl.ds(base + (NGC - 1) * GCH, 16)])


def kernel(xyzp, features, Wc, bc):
    y = jnp.clip(jnp.round(xyzp[..., 1] * H), 0, H - 1).astype(jnp.int32)
    x = jnp.clip(jnp.round(xyzp[..., 0] * W_), 0, W_ - 1).astype(jnp.int32)
    pos = xyzp[..., 3:4]
    rows = jnp.concatenate(
        [pos, 1.0 - pos, features, jnp.zeros((B, N, CS - CIN), jnp.float32)],
        axis=-1).astype(jnp.bfloat16)
    lkey = (y + 1) * Wp + (x + 1)

    rows_p = jnp.concatenate(
        [rows, jnp.zeros((B, 16 * PPW - N, CS), jnp.bfloat16)], axis=1)
    rows_p = rows_p.reshape(B * 16, 2, PPW // 2, CS)
    lkey_p = jnp.concatenate(
        [lkey, jnp.full((B, 16 * PPW - N), ZROW, jnp.int32)], axis=1)
    lkey_p = lkey_p.reshape(B * 16, 10, 128)

    g_grid = _sc_scatter(rows_p, lkey_p)

    wcat = jnp.pad(Wc, ((0, 0), (0, 32 - CIN), (0, 0))).reshape(9 * 32, COUT)
    wcat4 = jnp.concatenate(
        [jnp.pad(wcat, ((0, 0), (32 * q, 96 - 32 * q))) for q in range(4)], axis=0)
    bc8 = jnp.tile(bc[None, :], (8, 4))
    o_conv = _tc_conv(g_grid, wcat4, bc8).reshape(O_ROWS, COUT)

    gc = jnp.arange(B, dtype=jnp.int32)[:, None] * SB + lkey
    gkey = (4 * ((gc // 512) * 128 + gc % 128) + (gc // 128) % 4).reshape(32, PTS_W)
    gkey_p = jnp.concatenate(
        [gkey, jnp.zeros((32, NGC * GCH - PTS_W), jnp.int32)], axis=1)
    gkey_p = gkey_p.reshape(32, NGC, GCH)
    out = _sc_gather(o_conv, gkey_p)
    return out.reshape(B, N, COUT)
